# baseline (speedup 1.0000x reference)
import jax
import jax.numpy as jnp
import numpy as np
from jax.experimental import pallas as pl

NWAVE = 8
NORBIT = 64
OC_LOOP = 3
CUTOFF = 4.0
NUM_CLASSES = 118
_INDEX_PARA = np.array([0, 1, 1, 1])


def _ln(x):
    m = jnp.mean(x, axis=-1, keepdims=True)
    v = jnp.var(x, axis=-1, keepdims=True)
    return (x - m) / jnp.sqrt(v + 1e-5)


def _mlp2(x, params):
    for (W, b) in params[:-1]:
        x = x @ W + b
        x = _ln(x)
        x = jax.nn.silu(x)
    W, b = params[-1]
    return x @ W + b


def kernel(cart, atom_index, local_species, neigh_species, emb_neigh_params, emb_center_params, oc_params, out_params, contracted_coeff):
    nlocal = local_species.shape[0]
    flat = atom_index.reshape(-1)
    selected = cart[flat].reshape(2, -1, 3)
    dist_vec = selected[0] - selected[1]
    distances = jnp.linalg.norm(dist_vec, axis=-1)
    one_hot = jax.nn.one_hot(local_species + 1, NUM_CLASSES, dtype=jnp.float32)
    center_coeff = _mlp2(one_hot, emb_center_params)
    expand_spec = one_hot[flat].reshape(2, -1, NUM_CLASSES)
    hyper_spec = expand_spec[0] + expand_spec[1]
    neigh_emb = _mlp2(hyper_spec, emb_neigh_params).T
    cut_d = jnp.square(0.5 * jnp.cos(distances * (np.pi / CUTOFF)) + 0.5)
    radial = jnp.exp(-jnp.square(neigh_emb[NWAVE:2 * NWAVE] * (distances - neigh_emb[2 * NWAVE:3 * NWAVE])))
    angular = jnp.concatenate([cut_d[None, :], cut_d[None, :] * dist_vec.T], axis=0)
    orbital = jnp.einsum('ji,ki->ijk', angular, radial)
    weight_orbital = jnp.einsum('ijk,ki->ijk', orbital, neigh_emb[:NWAVE])
    zero_orbital = jnp.zeros((nlocal, angular.shape[0], NWAVE), dtype=cart.dtype)
    cc = contracted_coeff[:, jnp.asarray(_INDEX_PARA)]
    center_orbital = zero_orbital.at[atom_index[0]].add(weight_orbital)
    contracted = jnp.einsum('ijk,jkm->ijm', center_orbital, cc[0])
    density = jnp.einsum('ijm,ijm->im', contracted, contracted) + center_coeff
    iter_coeff = neigh_emb[:NWAVE].T
    neigh_list = atom_index[1]
    for i in range(OC_LOOP):
        nnout = _mlp2(density, oc_params[i])
        iter_coeff = iter_coeff + nnout[neigh_list]
        w_orb = jnp.einsum('ijk,ik->ijk', orbital, iter_coeff)
        center_orbital = zero_orbital.at[atom_index[0]].add(w_orb)
        contracted = jnp.einsum('ijk,jkm->ijm', center_orbital, cc[i + 1])
        density = jnp.einsum('ijm,ijm->im', contracted, contracted) + center_coeff
    mask = (local_species > -0.5).astype(cart.dtype)
    out1 = _mlp2(density, out_params)
    output = out1 * mask[:, None]
    return (dist_vec, output)


# speedup vs baseline: 38.5140x; 38.5140x over previous
import functools

import jax
import jax.numpy as jnp
import numpy as np
from jax import lax
from jax.experimental import pallas as pl
from jax.experimental.pallas import tpu as pltpu
from jax.experimental.pallas import tpu_sc as plsc

NWAVE = 8
NORBIT = 64
OC_LOOP = 3
CUTOFF = 4.0
NUM_CLASSES = 118
_IDXP = np.array([0, 1, 1, 1])

_NC = 2
_NS = 16
_NW = _NC * _NS
_B = 128


def _mesh():
    return plsc.VectorSubcoreMesh(
        core_axis_name="c", subcore_axis_name="s", num_cores=_NC,
        num_subcores=_NS)


def _sc_gather(tables, idxs):
    ntab = len(tables)
    nblk = idxs[0].shape[1]
    npts = _NW * nblk * _B
    outs = tuple(
        jax.ShapeDtypeStruct((npts, int(t.shape[1])), jnp.float32)
        for t in tables)
    scratch = []
    for t in tables:
        scratch += [
            pltpu.VMEM((nblk, _B), jnp.int32),
            pltpu.VMEM((_B, int(t.shape[1])), jnp.float32),
            pltpu.SemaphoreType.DMA,
        ]

    @functools.partial(pl.kernel, out_type=outs, mesh=_mesh(),
                       scratch_types=scratch,
                       compiler_params=pltpu.CompilerParams(
                           use_tc_tiling_on_sc=False))
    def k(*refs):
        tbl = refs[:ntab]
        idx = refs[ntab:2 * ntab]
        out = refs[2 * ntab:3 * ntab]
        sc = refs[3 * ntab:]
        wid = lax.axis_index("s") * _NC + lax.axis_index("c")
        for t in range(ntab):
            pltpu.sync_copy(idx[t].at[wid], sc[3 * t])

        def body(j, carry):
            base = wid * nblk * _B + j * _B
            cps = []
            for t in range(ntab):
                idx_v, rbuf, sem = sc[3 * t], sc[3 * t + 1], sc[3 * t + 2]
                cps.append(pltpu.async_copy(tbl[t].at[idx_v.at[j]], rbuf, sem))
            for t in range(ntab):
                rbuf = sc[3 * t + 1]
                cps[t].wait()
                pltpu.sync_copy(rbuf, out[t].at[pl.ds(base, _B)])
            return carry

        lax.fori_loop(0, nblk, body, 0)

    res = k(*tables, *idxs)
    if not isinstance(res, (list, tuple)):
        res = [res]
    return list(res)


def _sc_scatter_add(payload, idx2d, zrows, nout):
    nblk = idx2d.shape[1]
    epad = _NW * nblk * _B
    rpt = nout // _NS

    @functools.partial(
        pl.kernel,
        out_type=jax.ShapeDtypeStruct((_NC, nout, 32), jnp.float32),
        mesh=_mesh(),
        scratch_types=[
            pltpu.VMEM((nblk, _B), jnp.int32),
            pltpu.VMEM((_B, 32), jnp.float32),
            pltpu.VMEM_SHARED((nout, 32), jnp.float32),
        ],
        compiler_params=pltpu.CompilerParams(use_tc_tiling_on_sc=False))
    def k(p_hbm, i_hbm, z_hbm, out_hbm, idx_v, pbuf, acc):
        cid = lax.axis_index("c")
        sid = lax.axis_index("s")
        wid = sid * _NC + cid
        pltpu.sync_copy(z_hbm, acc.at[pl.ds(sid * rpt, rpt)])
        pltpu.sync_copy(i_hbm.at[wid], idx_v)
        plsc.subcore_barrier()

        def body(j, carry):
            base = wid * nblk * _B + j * _B
            pltpu.sync_copy(p_hbm.at[pl.ds(base, _B)], pbuf)
            pltpu.sync_copy(pbuf, acc.at[idx_v.at[j]], add=True)
            return carry

        lax.fori_loop(0, nblk, body, 0)
        plsc.subcore_barrier()
        pltpu.sync_copy(acc.at[pl.ds(sid * rpt, rpt)],
                        out_hbm.at[cid, pl.ds(sid * rpt, rpt)])

    return k(payload, idx2d, zrows)


def _ln_cols(x):
    m = jnp.mean(x, axis=0, keepdims=True)
    v = jnp.mean(jnp.square(x - m), axis=0, keepdims=True)
    return (x - m) / jnp.sqrt(v + 1e-5)


def _ln_rows(x):
    m = jnp.mean(x, axis=1, keepdims=True)
    v = jnp.mean(jnp.square(x - m), axis=1, keepdims=True)
    return (x - m) / jnp.sqrt(v + 1e-5)


def _silu(x):
    return x * jax.nn.sigmoid(x)


def _tc_center_table(w1p, b1, w2, b2):
    def body(w1_r, b1_r, w2_r, b2_r, o_r):
        x = w1_r[...] + b1_r[...]
        h = _silu(_ln_rows(x))
        o_r[...] = jnp.dot(h, w2_r[...],
                           preferred_element_type=jnp.float32, precision=lax.Precision.HIGHEST) + b2_r[...]

    return pl.pallas_call(
        body, out_shape=jax.ShapeDtypeStruct((128, NORBIT), jnp.float32),
    )(w1p, b1, w2, b2)


_BE = 2048


def _tc_edge_dense(ga0t, ga1t, w2t, b1, b2t):
    epad = ga0t.shape[1]

    def body(a_r, b_r, w2_r, b1_r, b2_r, dv_o, rad_o, co_o, ang_o, pay_o):
        a = a_r[...]
        b = b_r[...]
        dv = a[0:3, :] - b[0:3, :]
        d2 = jnp.sum(dv * dv, axis=0, keepdims=True)
        dist = jnp.sqrt(d2)
        e = a[8:16, :] + b[8:16, :] + b1_r[...]
        h = _silu(_ln_cols(e))
        ne = jnp.dot(w2_r[...], h,
                     preferred_element_type=jnp.float32, precision=lax.Precision.HIGHEST) + b2_r[...]
        w = ne[0:8, :]
        rad = jnp.exp(-jnp.square(ne[8:16, :] * (dist - ne[16:24, :])))
        cut = jnp.square(0.5 * jnp.cos(dist * (np.pi / CUTOFF)) + 0.5)
        ang = jnp.concatenate([cut, cut * dv], axis=0)
        dv_o[...] = jnp.concatenate([dv, dist], axis=0)
        rad_o[...] = rad
        co_o[...] = w
        ang_o[...] = ang
        rw = rad * w
        pay_o[...] = jnp.concatenate(
            [rw * ang[j:j + 1, :] for j in range(4)], axis=0)

    grid = (epad // _BE,)
    espec = lambda r: pl.BlockSpec((r, _BE), lambda i: (0, i))
    fspec = lambda shape: pl.BlockSpec(shape, lambda i: (0, 0))
    return pl.pallas_call(
        body,
        grid=grid,
        in_specs=[espec(16), espec(16), fspec((24, 8)), fspec((8, 1)),
                  fspec((24, 1))],
        out_specs=[espec(4), espec(8), espec(8), espec(4), espec(32)],
        out_shape=[
            jax.ShapeDtypeStruct((4, epad), jnp.float32),
            jax.ShapeDtypeStruct((8, epad), jnp.float32),
            jax.ShapeDtypeStruct((8, epad), jnp.float32),
            jax.ShapeDtypeStruct((4, epad), jnp.float32),
            jax.ShapeDtypeStruct((32, epad), jnp.float32),
        ],
    )(ga0t, ga1t, w2t, b1, b2t)


def _tc_payload_update(radt, angt, cot, jat):
    epad = radt.shape[1]

    def body(r_r, a_r, c_r, j_r, co_o, pay_o):
        c = c_r[...] + j_r[...]
        rw = r_r[...] * c
        ang = a_r[...]
        co_o[...] = c
        pay_o[...] = jnp.concatenate(
            [rw * ang[j:j + 1, :] for j in range(4)], axis=0)

    grid = (epad // _BE,)
    espec = lambda r: pl.BlockSpec((r, _BE), lambda i: (0, i))
    return pl.pallas_call(
        body,
        grid=grid,
        in_specs=[espec(8), espec(4), espec(8), espec(8)],
        out_specs=[espec(8), espec(32)],
        out_shape=[
            jax.ShapeDtypeStruct((8, epad), jnp.float32),
            jax.ShapeDtypeStruct((32, epad), jnp.float32),
        ],
    )(radt, angt, cot, jat)


_BN = 1024


def _tc_density_mlp(part, ccen, ccr, p0, p1, p2, spf, final):
    npad = ccen.shape[0]
    kk = p2[0].shape[1]

    def body(part_r, cc_r, ccr_r, v1_r, c1_r, v2_r, c2_r, v3_r, c3_r, sp_r,
             o_r):
        co = part_r[0] + part_r[1]
        acc = cc_r[...]
        for j in range(4):
            t = jnp.dot(co[:, 8 * j:8 * j + 8], ccr_r[8 * j:8 * j + 8, :],
                        preferred_element_type=jnp.float32, precision=lax.Precision.HIGHEST)
            acc = acc + t * t
        h = acc
        for (v_r, c_r) in ((v1_r, c1_r), (v2_r, c2_r)):
            h = jnp.dot(h, v_r[...],
                        preferred_element_type=jnp.float32, precision=lax.Precision.HIGHEST) + c_r[...]
            h = _silu(_ln_rows(h))
        o = jnp.dot(h, v3_r[...],
                    preferred_element_type=jnp.float32, precision=lax.Precision.HIGHEST) + c3_r[...]
        if final:
            o = o * (sp_r[...] > -0.5).astype(jnp.float32)
        o_r[...] = o

    grid = (npad // _BN,)
    fspec = lambda shape: pl.BlockSpec(shape, lambda i: tuple(0 for _ in shape))
    return pl.pallas_call(
        body,
        grid=grid,
        in_specs=[
            pl.BlockSpec((2, _BN, 32), lambda i: (0, i, 0)),
            pl.BlockSpec((_BN, 64), lambda i: (i, 0)),
            fspec((32, 64)),
            fspec((64, 64)), fspec((1, 64)),
            fspec((64, 64)), fspec((1, 64)),
            fspec((64, kk)), fspec((1, kk)),
            pl.BlockSpec((_BN, 1), lambda i: (i, 0)),
        ],
        out_specs=pl.BlockSpec((_BN, kk), lambda i: (i, 0)),
        out_shape=jax.ShapeDtypeStruct((npad, kk), jnp.float32),
    )(part, ccen, ccr, p0[0], p0[1], p1[0], p1[1], p2[0], p2[1], spf)


def kernel(cart, atom_index, local_species, neigh_species, emb_neigh_params,
           emb_center_params, oc_params, out_params, contracted_coeff):
    n = local_species.shape[0]
    e = atom_index.shape[1]
    npad = ((n + _NW * _B - 1) // (_NW * _B)) * (_NW * _B)
    epad = ((e + _NW * _B - 1) // (_NW * _B)) * (_NW * _B)

    f32 = jnp.float32
    sp = local_species.astype(jnp.int32) + 1
    sp_pad = jnp.concatenate([sp, jnp.zeros((npad - n,), jnp.int32)])
    sp2d = sp_pad.reshape(_NW, -1, _B)
    a0 = atom_index[0].astype(jnp.int32)
    a1 = atom_index[1].astype(jnp.int32)
    a0p = jnp.concatenate([a0, jnp.full((epad - e,), n, jnp.int32)])
    a1p = jnp.concatenate([a1, jnp.zeros((epad - e,), jnp.int32)])
    a0p2d = a0p.reshape(_NW, -1, _B)
    a1p2d = a1p.reshape(_NW, -1, _B)

    (w1n, b1n), (w2n, b2n) = emb_neigh_params
    (w1c, b1c), (w2c, b2c) = emb_center_params
    w1n_pad = jnp.concatenate(
        [w1n, jnp.zeros((128 - w1n.shape[0], w1n.shape[1]), f32)])
    w1c_pad = jnp.concatenate(
        [w1c, jnp.zeros((128 - w1c.shape[0], w1c.shape[1]), f32)])

    ctbl = _tc_center_table(w1c_pad, b1c.reshape(1, -1), w2c,
                            b2c.reshape(1, -1))
    emb8, ccen = _sc_gather([w1n_pad, ctbl], [sp2d, sp2d])

    nf = jnp.concatenate(
        [jnp.concatenate([cart, jnp.zeros((npad - n, 3), f32)], axis=0),
         jnp.zeros((npad, 5), f32), emb8], axis=1)

    ga0, ga1 = _sc_gather([nf, nf], [a0p2d, a1p2d])

    dvt, radt, cot, angt, payt = _tc_edge_dense(
        ga0.T, ga1.T, w2n.T, b1n.reshape(-1, 1), b2n.reshape(-1, 1))

    zrows = jnp.zeros((npad // _NS, 32), f32)
    ccr = contracted_coeff[:, jnp.asarray(_IDXP)]
    spf = sp_pad.astype(f32).reshape(-1, 1) - 1.0

    nn = None
    for r in range(OC_LOOP + 1):
        part = _sc_scatter_add(payt.T, a0p2d, zrows, npad)
        params = oc_params[r] if r < OC_LOOP else out_params
        nn = _tc_density_mlp(
            part, ccen, ccr[r].reshape(32, 64),
            (params[0][0], params[0][1].reshape(1, -1)),
            (params[1][0], params[1][1].reshape(1, -1)),
            (params[2][0], params[2][1].reshape(1, -1)),
            spf, final=(r == OC_LOOP))
        if r < OC_LOOP:
            (ja,) = _sc_gather([nn], [a1p2d])
            cot, payt = _tc_payload_update(radt, angt, cot, ja.T)

    dist_vec = dvt[:3, :e].T
    output = nn[:n]
    return (dist_vec, output)


# speedup vs baseline: 41.5885x; 1.0798x over previous
import functools

import jax
import jax.numpy as jnp
import numpy as np
from jax import lax
from jax.experimental import pallas as pl
from jax.experimental.pallas import tpu as pltpu
from jax.experimental.pallas import tpu_sc as plsc

NWAVE = 8
NORBIT = 64
OC_LOOP = 3
CUTOFF = 4.0
NUM_CLASSES = 118
_IDXP = np.array([0, 1, 1, 1])

_NC = 2
_NS = 16
_NW = _NC * _NS
_B = 128


def _mesh():
    return plsc.VectorSubcoreMesh(
        core_axis_name="c", subcore_axis_name="s", num_cores=_NC,
        num_subcores=_NS)


def _sc_gather(tables, idxs):
    ntab = len(tables)
    nblk = idxs[0].shape[1]
    npts = _NW * nblk * _B
    outs = tuple(
        jax.ShapeDtypeStruct((npts, int(t.shape[1])), jnp.float32)
        for t in tables)
    scratch = []
    for t in tables:
        scratch += [
            pltpu.VMEM((nblk, _B), jnp.int32),
            pltpu.VMEM((_B, int(t.shape[1])), jnp.float32),
            pltpu.SemaphoreType.DMA,
        ]

    @functools.partial(pl.kernel, out_type=outs, mesh=_mesh(),
                       scratch_types=scratch,
                       compiler_params=pltpu.CompilerParams(
                           use_tc_tiling_on_sc=False))
    def k(*refs):
        tbl = refs[:ntab]
        idx = refs[ntab:2 * ntab]
        out = refs[2 * ntab:3 * ntab]
        sc = refs[3 * ntab:]
        wid = lax.axis_index("s") * _NC + lax.axis_index("c")
        for t in range(ntab):
            pltpu.sync_copy(idx[t].at[wid], sc[3 * t])

        def body(j, carry):
            base = wid * nblk * _B + j * _B
            cps = []
            for t in range(ntab):
                idx_v, rbuf, sem = sc[3 * t], sc[3 * t + 1], sc[3 * t + 2]
                cps.append(pltpu.async_copy(tbl[t].at[idx_v.at[j]], rbuf, sem))
            for t in range(ntab):
                rbuf = sc[3 * t + 1]
                cps[t].wait()
                pltpu.sync_copy(rbuf, out[t].at[pl.ds(base, _B)])
            return carry

        lax.fori_loop(0, nblk, body, 0)

    res = k(*tables, *idxs)
    if not isinstance(res, (list, tuple)):
        res = [res]
    return list(res)


def _sc_round(radt, angt, cot, nnsum, idx0, idx1, zrows, nacc):
    nblk = idx0.shape[1]
    nnodes = nnsum.shape[0]
    rpt = nacc // _NS

    @functools.partial(
        pl.kernel,
        out_type=jax.ShapeDtypeStruct((_NC, nacc, 32), jnp.float32),
        mesh=_mesh(),
        scratch_types=[
            pltpu.VMEM((_B,), jnp.int32),
            pltpu.VMEM((_B,), jnp.int32),
            pltpu.VMEM((nnodes, 8), jnp.float32),
            pltpu.VMEM((8, _B), jnp.float32),
            pltpu.VMEM((4, _B), jnp.float32),
            pltpu.VMEM((8, _B), jnp.float32),
            pltpu.VMEM((_B, 32), jnp.float32),
            pltpu.VMEM_SHARED((nacc, 32), jnp.float32),
        ],
        compiler_params=pltpu.CompilerParams(
            use_tc_tiling_on_sc=False, needs_layout_passes=False))
    def k(rad_h, ang_h, co_h, nn_h, i0_h, i1_h, z_h, out_h,
          i0b, i1b, nn_v, rbuf, abuf, cbuf, pbuf, acc):
        cid = lax.axis_index("c")
        sid = lax.axis_index("s")
        wid = sid * _NC + cid
        pltpu.sync_copy(z_h, acc.at[pl.ds(sid * rpt, rpt)])
        pltpu.sync_copy(nn_h, nn_v)
        plsc.subcore_barrier()

        def body(j, carry):
            base = wid * nblk * _B + j * _B
            pltpu.sync_copy(i0_h.at[wid, j], i0b)
            pltpu.sync_copy(i1_h.at[wid, j], i1b)
            pltpu.sync_copy(rad_h.at[:, pl.ds(base, _B)], rbuf)
            pltpu.sync_copy(ang_h.at[:, pl.ds(base, _B)], abuf)
            pltpu.sync_copy(co_h.at[:, pl.ds(base, _B)], cbuf)
            for g in range(8):
                sl = pl.ds(g * 16, 16)
                e1 = i1b[sl]
                erows = lax.iota(jnp.int32, 16) + (g * 16)
                for kw in range(8):
                    nk = plsc.load_gather(
                        nn_v, [e1, jnp.full((16,), kw, jnp.int32)])
                    rk = rbuf[kw, sl] * (cbuf[kw, sl] + nk)
                    for ja in range(4):
                        plsc.store_scatter(
                            pbuf,
                            [erows, jnp.full((16,), ja * 8 + kw, jnp.int32)],
                            rk * abuf[ja, sl])
            pltpu.sync_copy(pbuf, acc.at[i0b], add=True)
            return carry

        lax.fori_loop(0, nblk, body, 0)
        plsc.subcore_barrier()
        pltpu.sync_copy(acc.at[pl.ds(sid * rpt, rpt)],
                        out_h.at[cid, pl.ds(sid * rpt, rpt)])

    return k(radt, angt, cot, nnsum, idx0, idx1, zrows)


def _ln_cols(x):
    m = jnp.mean(x, axis=0, keepdims=True)
    v = jnp.mean(jnp.square(x - m), axis=0, keepdims=True)
    return (x - m) / jnp.sqrt(v + 1e-5)


def _ln_rows(x):
    m = jnp.mean(x, axis=1, keepdims=True)
    v = jnp.mean(jnp.square(x - m), axis=1, keepdims=True)
    return (x - m) / jnp.sqrt(v + 1e-5)


def _silu(x):
    return x * jax.nn.sigmoid(x)


def _rb(x):
    return x.astype(jnp.bfloat16).astype(jnp.float32)


def _dotd(a, b):
    return jnp.dot(_rb(a), _rb(b), preferred_element_type=jnp.float32,
                   precision=lax.Precision.HIGHEST)


def _tc_center_table(w1p, b1, w2, b2):
    def body(w1_r, b1_r, w2_r, b2_r, o_r):
        x = _rb(w1_r[...]) + b1_r[...]
        h = _silu(_ln_rows(x))
        o_r[...] = _dotd(h, w2_r[...]) + b2_r[...]

    return pl.pallas_call(
        body, out_shape=jax.ShapeDtypeStruct((128, NORBIT), jnp.float32),
    )(w1p, b1, w2, b2)


_BE = 2048


def _tc_edge_dense(ga0t, ga1t, w2t, b1, b2t):
    epad = ga0t.shape[1]

    def body(a_r, b_r, w2_r, b1_r, b2_r, dv_o, rad_o, co_o, ang_o):
        a = a_r[...]
        b = b_r[...]
        dv = a[0:3, :] - b[0:3, :]
        d2 = jnp.sum(dv * dv, axis=0, keepdims=True)
        dist = jnp.sqrt(d2)
        e = a[8:16, :] + b[8:16, :] + b1_r[...]
        h = _silu(_ln_cols(e))
        ne = _dotd(w2_r[...], h) + b2_r[...]
        w = ne[0:8, :]
        rad = jnp.exp(-jnp.square(ne[8:16, :] * (dist - ne[16:24, :])))
        cut = jnp.square(0.5 * jnp.cos(dist * (np.pi / CUTOFF)) + 0.5)
        ang = jnp.concatenate([cut, cut * dv], axis=0)
        dv_o[...] = jnp.concatenate([dv, dist], axis=0)
        rad_o[...] = rad
        co_o[...] = w
        ang_o[...] = ang

    grid = (epad // _BE,)
    espec = lambda r: pl.BlockSpec((r, _BE), lambda i: (0, i))
    fspec = lambda shape: pl.BlockSpec(shape, lambda i: (0, 0))
    return pl.pallas_call(
        body,
        grid=grid,
        in_specs=[espec(16), espec(16), fspec((24, 8)), fspec((8, 1)),
                  fspec((24, 1))],
        out_specs=[espec(4), espec(8), espec(8), espec(4)],
        out_shape=[
            jax.ShapeDtypeStruct((4, epad), jnp.float32),
            jax.ShapeDtypeStruct((8, epad), jnp.float32),
            jax.ShapeDtypeStruct((8, epad), jnp.float32),
            jax.ShapeDtypeStruct((4, epad), jnp.float32),
        ],
    )(ga0t, ga1t, w2t, b1, b2t)


_BN = 1024


def _tc_density_mlp(part, ccen, ccr, p0, p1, p2, spf, nnsum, final):
    npad = ccen.shape[0]
    kk = p2[0].shape[1]
    pblk_max = part.shape[1] // _BN - 1

    def body(part_r, cc_r, ccr_r, v1_r, c1_r, v2_r, c2_r, v3_r, c3_r, sp_r,
             ns_r, o_r):
        co = part_r[0] + part_r[1]
        acc = cc_r[...]
        for j in range(4):
            t = _dotd(co[:, 8 * j:8 * j + 8], ccr_r[8 * j:8 * j + 8, :])
            acc = acc + t * t
        h = acc
        for (v_r, c_r) in ((v1_r, c1_r), (v2_r, c2_r)):
            h = _dotd(h, v_r[...]) + c_r[...]
            h = _silu(_ln_rows(h))
        o = _dotd(h, v3_r[...]) + c3_r[...]
        if final:
            o = o * (sp_r[...] > -0.5).astype(jnp.float32)
        else:
            o = o + ns_r[...]
        o_r[...] = o

    grid = (npad // _BN,)
    fspec = lambda shape: pl.BlockSpec(shape, lambda i: tuple(0 for _ in shape))
    return pl.pallas_call(
        body,
        grid=grid,
        in_specs=[
            pl.BlockSpec((2, _BN, 32),
                         lambda i: (0, jnp.minimum(i, pblk_max), 0)),
            pl.BlockSpec((_BN, 64), lambda i: (i, 0)),
            fspec((32, 64)),
            fspec((64, 64)), fspec((1, 64)),
            fspec((64, 64)), fspec((1, 64)),
            fspec((64, kk)), fspec((1, kk)),
            pl.BlockSpec((_BN, 1), lambda i: (i, 0)),
            pl.BlockSpec((_BN, 8), lambda i: (i, 0)),
        ],
        out_specs=pl.BlockSpec((_BN, kk), lambda i: (i, 0)),
        out_shape=jax.ShapeDtypeStruct((npad, kk), jnp.float32),
    )(part, ccen, ccr, p0[0], p0[1], p1[0], p1[1], p2[0], p2[1], spf, nnsum)


def kernel(cart, atom_index, local_species, neigh_species, emb_neigh_params,
           emb_center_params, oc_params, out_params, contracted_coeff):
    n = local_species.shape[0]
    e = atom_index.shape[1]
    npad = ((n + _NW * _B - 1) // (_NW * _B)) * (_NW * _B)
    epad = ((e + _NW * _B - 1) // (_NW * _B)) * (_NW * _B)

    f32 = jnp.float32
    sp = local_species.astype(jnp.int32) + 1
    sp_pad = jnp.concatenate([sp, jnp.zeros((npad - n,), jnp.int32)])
    sp2d = sp_pad.reshape(_NW, -1, _B)
    a0 = atom_index[0].astype(jnp.int32)
    a1 = atom_index[1].astype(jnp.int32)
    a0p = jnp.concatenate([a0, jnp.full((epad - e,), n, jnp.int32)])
    a1p = jnp.concatenate([a1, jnp.zeros((epad - e,), jnp.int32)])
    a0p2d = a0p.reshape(_NW, -1, _B)
    a1p2d = a1p.reshape(_NW, -1, _B)

    (w1n, b1n), (w2n, b2n) = emb_neigh_params
    (w1c, b1c), (w2c, b2c) = emb_center_params
    w1n_pad = jnp.concatenate(
        [w1n.astype(jnp.bfloat16).astype(f32),
         jnp.zeros((128 - w1n.shape[0], w1n.shape[1]), f32)])
    w1c_pad = jnp.concatenate(
        [w1c, jnp.zeros((128 - w1c.shape[0], w1c.shape[1]), f32)])

    ctbl = _tc_center_table(w1c_pad, b1c.reshape(1, -1), w2c,
                            b2c.reshape(1, -1))
    emb8, ccen = _sc_gather([w1n_pad, ctbl], [sp2d, sp2d])

    nf = jnp.concatenate(
        [jnp.concatenate([cart, jnp.zeros((npad - n, 3), f32)], axis=0),
         jnp.zeros((npad, 5), f32), emb8], axis=1)

    ga0, ga1 = _sc_gather([nf, nf], [a0p2d, a1p2d])

    dvt, radt, cot, angt = _tc_edge_dense(
        ga0.T, ga1.T, w2n.T, b1n.reshape(-1, 1), b2n.reshape(-1, 1))

    nacc = 10240
    zrows = jnp.zeros((nacc // _NS, 32), f32)
    ccr = contracted_coeff[:, jnp.asarray(_IDXP)]
    spf = sp_pad.astype(f32).reshape(-1, 1) - 1.0

    nn = jnp.zeros((npad, 8), f32)
    for r in range(OC_LOOP + 1):
        part = _sc_round(radt, angt, cot, nn, a0p2d, a1p2d, zrows, nacc)
        params = oc_params[r] if r < OC_LOOP else out_params
        nn = _tc_density_mlp(
            part, ccen, ccr[r].reshape(32, 64),
            (params[0][0], params[0][1].reshape(1, -1)),
            (params[1][0], params[1][1].reshape(1, -1)),
            (params[2][0], params[2][1].reshape(1, -1)),
            spf, nn, final=(r == OC_LOOP))

    dist_vec = dvt[:3, :e].T
    output = nn[:n]
    return (dist_vec, output)


# speedup vs baseline: 52.9458x; 1.2731x over previous
import functools

import jax
import jax.numpy as jnp
import numpy as np
from jax import lax
from jax.experimental import pallas as pl
from jax.experimental.pallas import tpu as pltpu
from jax.experimental.pallas import tpu_sc as plsc

NWAVE = 8
NORBIT = 64
OC_LOOP = 3
CUTOFF = 4.0
NUM_CLASSES = 118
_IDXP = np.array([0, 1, 1, 1])

_NC = 2
_NS = 16
_NW = _NC * _NS
_B = 128


def _mesh():
    return plsc.VectorSubcoreMesh(
        core_axis_name="c", subcore_axis_name="s", num_cores=_NC,
        num_subcores=_NS)


def _sc_gather(tables, idxs):
    ntab = len(tables)
    nblk = idxs[0].shape[1]
    npts = _NW * nblk * _B
    outs = tuple(
        jax.ShapeDtypeStruct((npts, int(t.shape[1])), jnp.float32)
        for t in tables)
    scratch = []
    for t in tables:
        scratch += [
            pltpu.VMEM((nblk, _B), jnp.int32),
            pltpu.VMEM((_B, int(t.shape[1])), jnp.float32),
            pltpu.SemaphoreType.DMA,
        ]

    @functools.partial(pl.kernel, out_type=outs, mesh=_mesh(),
                       scratch_types=scratch,
                       compiler_params=pltpu.CompilerParams(
                           use_tc_tiling_on_sc=False))
    def k(*refs):
        tbl = refs[:ntab]
        idx = refs[ntab:2 * ntab]
        out = refs[2 * ntab:3 * ntab]
        sc = refs[3 * ntab:]
        wid = lax.axis_index("s") * _NC + lax.axis_index("c")
        for t in range(ntab):
            pltpu.sync_copy(idx[t].at[wid], sc[3 * t])

        def body(j, carry):
            base = wid * nblk * _B + j * _B
            cps = []
            for t in range(ntab):
                idx_v, rbuf, sem = sc[3 * t], sc[3 * t + 1], sc[3 * t + 2]
                cps.append(pltpu.async_copy(tbl[t].at[idx_v.at[j]], rbuf, sem))
            for t in range(ntab):
                rbuf = sc[3 * t + 1]
                cps[t].wait()
                pltpu.sync_copy(rbuf, out[t].at[pl.ds(base, _B)])
            return carry

        lax.fori_loop(0, nblk, body, 0)

    res = k(*tables, *idxs)
    if not isinstance(res, (list, tuple)):
        res = [res]
    return list(res)


def _sc_round(feat, nnsum, idx0, idx1, zrows, nacc):
    nblk = idx0.shape[1]
    rpt = nacc // _NS
    epad = feat.shape[1]

    @functools.partial(
        pl.kernel,
        out_type=jax.ShapeDtypeStruct((_NC, nacc, 32), jnp.float32),
        mesh=_mesh(),
        scratch_types=[
            pltpu.VMEM((nblk, _B), jnp.int32),
            pltpu.VMEM((nblk, _B), jnp.int32),
            pltpu.VMEM((nacc, 8), jnp.float32),
            pltpu.VMEM((20, _B), jnp.float32),
            pltpu.VMEM((20, _B), jnp.float32),
            pltpu.VMEM((_B, 32), jnp.float32),
            pltpu.SemaphoreType.DMA,
            pltpu.SemaphoreType.DMA,
            pltpu.VMEM_SHARED((nacc, 32), jnp.float32),
        ],
        compiler_params=pltpu.CompilerParams(
            use_tc_tiling_on_sc=False, needs_layout_passes=False))
    def k(f_h, nn_h, i0_h, i1_h, z_h, out_h,
          i0_v, i1_v, nn_v, fb0, fb1, pbuf, sl0, sl1, acc):
        cid = lax.axis_index("c")
        sid = lax.axis_index("s")
        wid = sid * _NC + cid
        ebase = wid * nblk * _B

        def start_load(j, fb, sem):
            pltpu.async_copy(f_h.at[:, pl.ds(ebase + j * _B, _B)], fb, sem)

        def wait_load(j, fb, sem):
            pltpu.make_async_copy(
                f_h.at[:, pl.ds(ebase + j * _B, _B)], fb, sem).wait()

        def compute_scatter(j, fb):
            for g in range(8):
                sl = pl.ds(g * 16, 16)
                e1 = i1_v[j, sl]
                erows = lax.iota(jnp.int32, 16) + (g * 16)
                for kw in range(8):
                    nk = plsc.load_gather(
                        nn_v, [e1, jnp.full((16,), kw, jnp.int32)])
                    rk = fb[kw, sl] * (fb[12 + kw, sl] + nk)
                    for ja in range(4):
                        plsc.store_scatter(
                            pbuf,
                            [erows, jnp.full((16,), ja * 8 + kw, jnp.int32)],
                            rk * fb[8 + ja, sl])
            pltpu.sync_copy(pbuf, acc.at[i0_v.at[j]], add=True)

        pltpu.sync_copy(i0_h.at[wid], i0_v)
        pltpu.sync_copy(i1_h.at[wid], i1_v)
        start_load(0, fb0, sl0)
        pltpu.sync_copy(z_h, acc.at[pl.ds(sid * rpt, rpt)])
        pltpu.sync_copy(nn_h.at[pl.ds(0, nacc)], nn_v)
        plsc.subcore_barrier()

        def body(p, carry):
            j0 = 2 * p
            j1 = 2 * p + 1
            start_load(j1, fb1, sl1)
            wait_load(j0, fb0, sl0)
            compute_scatter(j0, fb0)
            start_load(jnp.minimum(j0 + 2, nblk - 1), fb0, sl0)
            wait_load(j1, fb1, sl1)
            compute_scatter(j1, fb1)
            return carry

        lax.fori_loop(0, nblk // 2, body, 0)
        wait_load(nblk - 1, fb0, sl0)
        plsc.subcore_barrier()
        pltpu.sync_copy(acc.at[pl.ds(sid * rpt, rpt)],
                        out_h.at[cid, pl.ds(sid * rpt, rpt)])

    return k(feat, nnsum, idx0, idx1, zrows)


def _ln_cols(x):
    m = jnp.mean(x, axis=0, keepdims=True)
    v = jnp.mean(jnp.square(x - m), axis=0, keepdims=True)
    return (x - m) / jnp.sqrt(v + 1e-5)


def _ln_rows(x):
    m = jnp.mean(x, axis=1, keepdims=True)
    v = jnp.mean(jnp.square(x - m), axis=1, keepdims=True)
    return (x - m) / jnp.sqrt(v + 1e-5)


def _silu(x):
    return x * jax.nn.sigmoid(x)


def _rb(x):
    return x.astype(jnp.bfloat16).astype(jnp.float32)


def _dotd(a, b):
    return jnp.dot(_rb(a), _rb(b), preferred_element_type=jnp.float32,
                   precision=lax.Precision.HIGHEST)


def _tc_center_table(w1p, b1, w2, b2):
    def body(w1_r, b1_r, w2_r, b2_r, o_r):
        x = _rb(w1_r[...]) + b1_r[...]
        h = _silu(_ln_rows(x))
        o_r[...] = _dotd(h, w2_r[...]) + b2_r[...]

    return pl.pallas_call(
        body, out_shape=jax.ShapeDtypeStruct((128, NORBIT), jnp.float32),
    )(w1p, b1, w2, b2)


_BE = 2048


def _tc_edge_dense(ga0t, ga1t, w2t, b1, b2t):
    epad = ga0t.shape[1]

    def body(a_r, b_r, w2_r, b1_r, b2_r, dv_o, f_o):
        a = a_r[...]
        b = b_r[...]
        dv = a[0:3, :] - b[0:3, :]
        d2 = jnp.sum(dv * dv, axis=0, keepdims=True)
        dist = jnp.sqrt(d2)
        e = a[8:16, :] + b[8:16, :] + b1_r[...]
        h = _silu(_ln_cols(e))
        ne = _dotd(w2_r[...], h) + b2_r[...]
        w = ne[0:8, :]
        rad = jnp.exp(-jnp.square(ne[8:16, :] * (dist - ne[16:24, :])))
        cut = jnp.square(0.5 * jnp.cos(dist * (np.pi / CUTOFF)) + 0.5)
        ang = jnp.concatenate([cut, cut * dv], axis=0)
        dv_o[...] = jnp.concatenate([dv, dist], axis=0)
        f_o[...] = jnp.concatenate([rad, ang, w], axis=0)

    grid = (epad // _BE,)
    espec = lambda r: pl.BlockSpec((r, _BE), lambda i: (0, i))
    fspec = lambda shape: pl.BlockSpec(shape, lambda i: (0, 0))
    return pl.pallas_call(
        body,
        grid=grid,
        in_specs=[espec(16), espec(16), fspec((24, 8)), fspec((8, 1)),
                  fspec((24, 1))],
        out_specs=[espec(4), espec(20)],
        out_shape=[
            jax.ShapeDtypeStruct((4, epad), jnp.float32),
            jax.ShapeDtypeStruct((20, epad), jnp.float32),
        ],
    )(ga0t, ga1t, w2t, b1, b2t)


_BN = 1024


def _tc_density_mlp(part, ccen, ccr, p0, p1, p2, spf, nnsum, final):
    npad = ccen.shape[0]
    kk = p2[0].shape[1]
    pblk_max = part.shape[1] // _BN - 1

    def body(part_r, cc_r, ccr_r, v1_r, c1_r, v2_r, c2_r, v3_r, c3_r, sp_r,
             ns_r, o_r):
        co = part_r[0] + part_r[1]
        acc = cc_r[...]
        for j in range(4):
            t = _dotd(co[:, 8 * j:8 * j + 8], ccr_r[8 * j:8 * j + 8, :])
            acc = acc + t * t
        h = acc
        for (v_r, c_r) in ((v1_r, c1_r), (v2_r, c2_r)):
            h = _dotd(h, v_r[...]) + c_r[...]
            h = _silu(_ln_rows(h))
        o = _dotd(h, v3_r[...]) + c3_r[...]
        if final:
            o = o * (sp_r[...] > -0.5).astype(jnp.float32)
        else:
            o = o + ns_r[...]
        o_r[...] = o

    grid = (npad // _BN,)
    fspec = lambda shape: pl.BlockSpec(shape, lambda i: tuple(0 for _ in shape))
    return pl.pallas_call(
        body,
        grid=grid,
        in_specs=[
            pl.BlockSpec((2, _BN, 32),
                         lambda i: (0, jnp.minimum(i, pblk_max), 0)),
            pl.BlockSpec((_BN, 64), lambda i: (i, 0)),
            fspec((32, 64)),
            fspec((64, 64)), fspec((1, 64)),
            fspec((64, 64)), fspec((1, 64)),
            fspec((64, kk)), fspec((1, kk)),
            pl.BlockSpec((_BN, 1), lambda i: (i, 0)),
            pl.BlockSpec((_BN, 8), lambda i: (i, 0)),
        ],
        out_specs=pl.BlockSpec((_BN, kk), lambda i: (i, 0)),
        out_shape=jax.ShapeDtypeStruct((npad, kk), jnp.float32),
    )(part, ccen, ccr, p0[0], p0[1], p1[0], p1[1], p2[0], p2[1], spf, nnsum)


def kernel(cart, atom_index, local_species, neigh_species, emb_neigh_params,
           emb_center_params, oc_params, out_params, contracted_coeff):
    n = local_species.shape[0]
    e = atom_index.shape[1]
    npad = ((n + _NW * _B - 1) // (_NW * _B)) * (_NW * _B)
    epad = ((e + _NW * _B - 1) // (_NW * _B)) * (_NW * _B)

    f32 = jnp.float32
    sp = local_species.astype(jnp.int32) + 1
    sp_pad = jnp.concatenate([sp, jnp.zeros((npad - n,), jnp.int32)])
    sp2d = sp_pad.reshape(_NW, -1, _B)
    a0 = atom_index[0].astype(jnp.int32)
    a1 = atom_index[1].astype(jnp.int32)
    a0p = jnp.concatenate([a0, jnp.full((epad - e,), n, jnp.int32)])
    a1p = jnp.concatenate([a1, jnp.zeros((epad - e,), jnp.int32)])
    a0p2d = a0p.reshape(_NW, -1, _B)
    a1p2d = a1p.reshape(_NW, -1, _B)

    (w1n, b1n), (w2n, b2n) = emb_neigh_params
    (w1c, b1c), (w2c, b2c) = emb_center_params
    w1n_pad = jnp.concatenate(
        [w1n.astype(jnp.bfloat16).astype(f32),
         jnp.zeros((128 - w1n.shape[0], w1n.shape[1]), f32)])
    w1c_pad = jnp.concatenate(
        [w1c, jnp.zeros((128 - w1c.shape[0], w1c.shape[1]), f32)])

    ctbl = _tc_center_table(w1c_pad, b1c.reshape(1, -1), w2c,
                            b2c.reshape(1, -1))
    emb8, ccen = _sc_gather([w1n_pad, ctbl], [sp2d, sp2d])

    nf = jnp.concatenate(
        [jnp.concatenate([cart, jnp.zeros((npad - n, 3), f32)], axis=0),
         jnp.zeros((npad, 5), f32), emb8], axis=1)

    ga0, ga1 = _sc_gather([nf, nf], [a0p2d, a1p2d])

    dvt, feat = _tc_edge_dense(
        ga0.T, ga1.T, w2n.T, b1n.reshape(-1, 1), b2n.reshape(-1, 1))

    nacc = 10240
    zrows = jnp.zeros((nacc // _NS, 32), f32)
    ccr = contracted_coeff[:, jnp.asarray(_IDXP)]
    spf = sp_pad.astype(f32).reshape(-1, 1) - 1.0

    nn = jnp.zeros((npad, 8), f32)
    for r in range(OC_LOOP + 1):
        part = _sc_round(feat, nn, a0p2d, a1p2d, zrows, nacc)
        params = oc_params[r] if r < OC_LOOP else out_params
        nn = _tc_density_mlp(
            part, ccen, ccr[r].reshape(32, 64),
            (params[0][0], params[0][1].reshape(1, -1)),
            (params[1][0], params[1][1].reshape(1, -1)),
            (params[2][0], params[2][1].reshape(1, -1)),
            spf, nn, final=(r == OC_LOOP))

    dist_vec = dvt[:3, :e].T
    output = nn[:n]
    return (dist_vec, output)


# speedup vs baseline: 54.9440x; 1.0377x over previous
import functools

import jax
import jax.numpy as jnp
import numpy as np
from jax import lax
from jax.experimental import pallas as pl
from jax.experimental.pallas import tpu as pltpu
from jax.experimental.pallas import tpu_sc as plsc

NWAVE = 8
NORBIT = 64
OC_LOOP = 3
CUTOFF = 4.0
NUM_CLASSES = 118
_IDXP = np.array([0, 1, 1, 1])

_NC = 2
_NS = 16
_NW = _NC * _NS
_B = 128


def _mesh():
    return plsc.VectorSubcoreMesh(
        core_axis_name="c", subcore_axis_name="s", num_cores=_NC,
        num_subcores=_NS)


def _sc_gather(tables, idxs):
    ntab = len(tables)
    nblk = idxs[0].shape[1]
    npts = _NW * nblk * _B
    outs = tuple(
        jax.ShapeDtypeStruct((npts, int(t.shape[1])), jnp.float32)
        for t in tables)
    scratch = []
    for t in tables:
        scratch += [
            pltpu.VMEM((nblk, _B), jnp.int32),
            pltpu.VMEM((_B, int(t.shape[1])), jnp.float32),
            pltpu.SemaphoreType.DMA,
        ]

    @functools.partial(pl.kernel, out_type=outs, mesh=_mesh(),
                       scratch_types=scratch,
                       compiler_params=pltpu.CompilerParams(
                           use_tc_tiling_on_sc=False))
    def k(*refs):
        tbl = refs[:ntab]
        idx = refs[ntab:2 * ntab]
        out = refs[2 * ntab:3 * ntab]
        sc = refs[3 * ntab:]
        wid = lax.axis_index("s") * _NC + lax.axis_index("c")
        for t in range(ntab):
            pltpu.sync_copy(idx[t].at[wid], sc[3 * t])

        def body(j, carry):
            base = wid * nblk * _B + j * _B
            cps = []
            for t in range(ntab):
                idx_v, rbuf, sem = sc[3 * t], sc[3 * t + 1], sc[3 * t + 2]
                cps.append(pltpu.async_copy(tbl[t].at[idx_v.at[j]], rbuf, sem))
            for t in range(ntab):
                rbuf = sc[3 * t + 1]
                cps[t].wait()
                pltpu.sync_copy(rbuf, out[t].at[pl.ds(base, _B)])
            return carry

        lax.fori_loop(0, nblk, body, 0)

    res = k(*tables, *idxs)
    if not isinstance(res, (list, tuple)):
        res = [res]
    return list(res)


def _sc_round(feat, nnsum, idx0, idx1, zrows, nacc):
    nblk = idx0.shape[1]
    rpt = nacc // _NS
    epad = feat.shape[1]

    @functools.partial(
        pl.kernel,
        out_type=jax.ShapeDtypeStruct((_NC, nacc, 32), jnp.float32),
        mesh=_mesh(),
        scratch_types=[
            pltpu.VMEM((nblk, _B), jnp.int32),
            pltpu.VMEM((nblk, _B), jnp.int32),
            pltpu.VMEM((nacc, 8), jnp.float32),
            pltpu.VMEM((20, _B), jnp.float32),
            pltpu.VMEM((20, _B), jnp.float32),
            pltpu.VMEM((_B, 32), jnp.float32),
            pltpu.VMEM((_B, 32), jnp.float32),
            pltpu.SemaphoreType.DMA,
            pltpu.SemaphoreType.DMA,
            pltpu.SemaphoreType.DMA,
            pltpu.SemaphoreType.DMA,
            pltpu.VMEM_SHARED((nacc, 32), jnp.float32),
        ],
        compiler_params=pltpu.CompilerParams(
            use_tc_tiling_on_sc=False, needs_layout_passes=False))
    def k(f_h, nn_h, i0_h, i1_h, z_h, out_h,
          i0_v, i1_v, nn_v, fb0, fb1, pb0, pb1, sl0, sl1, ss0, ss1, acc):
        cid = lax.axis_index("c")
        sid = lax.axis_index("s")
        wid = sid * _NC + cid
        ebase = wid * nblk * _B

        def start_load(j, fb, sem):
            pltpu.async_copy(f_h.at[:, pl.ds(ebase + j * _B, _B)], fb, sem)

        def wait_load(j, fb, sem):
            pltpu.make_async_copy(
                f_h.at[:, pl.ds(ebase + j * _B, _B)], fb, sem).wait()

        def compute(j, fb, pb):
            for g in range(8):
                sl = pl.ds(g * 16, 16)
                e1 = i1_v[j, sl]
                erows = lax.iota(jnp.int32, 16) + (g * 16)
                for kw in range(8):
                    nk = plsc.load_gather(
                        nn_v, [e1, jnp.full((16,), kw, jnp.int32)])
                    rk = fb[kw, sl] * (fb[12 + kw, sl] + nk)
                    for ja in range(4):
                        plsc.store_scatter(
                            pb,
                            [erows, jnp.full((16,), ja * 8 + kw, jnp.int32)],
                            rk * fb[8 + ja, sl])

        def start_scatter(j, pb, sem):
            pltpu.async_copy(pb, acc.at[i0_v.at[j]], sem, add=True)

        def wait_scatter(j, pb, sem):
            pltpu.make_async_copy(pb, acc.at[i0_v.at[j]], sem).wait()

        pltpu.sync_copy(i0_h.at[wid], i0_v)
        pltpu.sync_copy(i1_h.at[wid], i1_v)
        start_load(0, fb0, sl0)
        pltpu.sync_copy(z_h, acc.at[pl.ds(sid * rpt, rpt)])
        pltpu.sync_copy(nn_h.at[pl.ds(0, nacc)], nn_v)
        plsc.subcore_barrier()

        def body(p, carry):
            j0 = 2 * p
            j1 = 2 * p + 1
            start_load(j1, fb1, sl1)
            wait_load(j0, fb0, sl0)

            @pl.when(p > 0)
            def _():
                wait_scatter(j0 - 2, pb0, ss0)
            compute(j0, fb0, pb0)
            start_scatter(j0, pb0, ss0)
            start_load(jnp.minimum(j0 + 2, nblk - 1), fb0, sl0)
            wait_load(j1, fb1, sl1)

            @pl.when(p > 0)
            def _():
                wait_scatter(j1 - 2, pb1, ss1)
            compute(j1, fb1, pb1)
            start_scatter(j1, pb1, ss1)
            return carry

        lax.fori_loop(0, nblk // 2, body, 0)
        wait_load(nblk - 1, fb0, sl0)
        wait_scatter(nblk - 2, pb0, ss0)
        wait_scatter(nblk - 1, pb1, ss1)
        plsc.subcore_barrier()
        pltpu.sync_copy(acc.at[pl.ds(sid * rpt, rpt)],
                        out_h.at[cid, pl.ds(sid * rpt, rpt)])

    return k(feat, nnsum, idx0, idx1, zrows)


def _ln_cols(x):
    m = jnp.mean(x, axis=0, keepdims=True)
    v = jnp.mean(jnp.square(x - m), axis=0, keepdims=True)
    return (x - m) / jnp.sqrt(v + 1e-5)


def _ln_rows(x):
    m = jnp.mean(x, axis=1, keepdims=True)
    v = jnp.mean(jnp.square(x - m), axis=1, keepdims=True)
    return (x - m) / jnp.sqrt(v + 1e-5)


def _silu(x):
    return x * jax.nn.sigmoid(x)


def _rb(x):
    return x.astype(jnp.bfloat16).astype(jnp.float32)


def _dotd(a, b):
    return jnp.dot(_rb(a), _rb(b), preferred_element_type=jnp.float32,
                   precision=lax.Precision.HIGHEST)


_BN = 1024


def _tc_node_feats(spi, cart4, w1nr, w1c, b1c, w2c, b2c):
    npad = spi.shape[0]

    def body(sp_r, c_r, w1n_r, w1c_r, b1c_r, w2c_r, b2c_r, nf_o, cc_o):
        ids = lax.broadcasted_iota(jnp.int32, (_BN, 128), 1)
        oh = (ids == sp_r[...]).astype(jnp.float32)
        emb8 = jnp.dot(oh, w1n_r[...], preferred_element_type=jnp.float32,
                       precision=lax.Precision.HIGHEST)
        htbl = _silu(_ln_rows(_rb(w1c_r[...]) + b1c_r[...]))
        ctbl = _dotd(htbl, w2c_r[...]) + b2c_r[...]
        cc_o[...] = jnp.dot(oh, ctbl, preferred_element_type=jnp.float32,
                            precision=lax.Precision.HIGHEST)
        nf_o[...] = jnp.concatenate(
            [c_r[...], jnp.zeros((_BN, 4), jnp.float32), emb8], axis=1)

    grid = (npad // _BN,)
    fspec = lambda shape: pl.BlockSpec(shape, lambda i: tuple(0 for _ in shape))
    return pl.pallas_call(
        body,
        grid=grid,
        in_specs=[
            pl.BlockSpec((_BN, 1), lambda i: (i, 0)),
            pl.BlockSpec((_BN, 4), lambda i: (i, 0)),
            fspec((128, 8)), fspec((128, 8)), fspec((1, 8)),
            fspec((8, NORBIT)), fspec((1, NORBIT)),
        ],
        out_specs=[pl.BlockSpec((_BN, 16), lambda i: (i, 0)),
                   pl.BlockSpec((_BN, NORBIT), lambda i: (i, 0))],
        out_shape=[jax.ShapeDtypeStruct((npad, 16), jnp.float32),
                   jax.ShapeDtypeStruct((npad, NORBIT), jnp.float32)],
    )(spi, cart4, w1nr, w1c, b1c, w2c, b2c)


_BE = 2048


def _tc_edge_dense(ga0t, ga1t, w2t, b1, b2t):
    epad = ga0t.shape[1]

    def body(a_r, b_r, w2_r, b1_r, b2_r, dv_o, f_o):
        a = a_r[...]
        b = b_r[...]
        dv = a[0:3, :] - b[0:3, :]
        d2 = jnp.sum(dv * dv, axis=0, keepdims=True)
        dist = jnp.sqrt(d2)
        e = a[8:16, :] + b[8:16, :] + b1_r[...]
        h = _silu(_ln_cols(e))
        ne = _dotd(w2_r[...], h) + b2_r[...]
        w = ne[0:8, :]
        rad = jnp.exp(-jnp.square(ne[8:16, :] * (dist - ne[16:24, :])))
        cut = jnp.square(0.5 * jnp.cos(dist * (np.pi / CUTOFF)) + 0.5)
        ang = jnp.concatenate([cut, cut * dv], axis=0)
        dv_o[...] = jnp.concatenate([dv, dist], axis=0)
        f_o[...] = jnp.concatenate([rad, ang, w], axis=0)

    grid = (epad // _BE,)
    espec = lambda r: pl.BlockSpec((r, _BE), lambda i: (0, i))
    fspec = lambda shape: pl.BlockSpec(shape, lambda i: (0, 0))
    return pl.pallas_call(
        body,
        grid=grid,
        in_specs=[espec(16), espec(16), fspec((24, 8)), fspec((8, 1)),
                  fspec((24, 1))],
        out_specs=[espec(4), espec(20)],
        out_shape=[
            jax.ShapeDtypeStruct((4, epad), jnp.float32),
            jax.ShapeDtypeStruct((20, epad), jnp.float32),
        ],
    )(ga0t, ga1t, w2t, b1, b2t)


def _tc_density_mlp(part, ccen, ccr, p0, p1, p2, spf, nnsum, final):
    npad = ccen.shape[0]
    kk = p2[0].shape[1]
    pblk_max = part.shape[1] // _BN - 1

    def body(part_r, cc_r, ccr_r, v1_r, c1_r, v2_r, c2_r, v3_r, c3_r, sp_r,
             ns_r, o_r):
        co = part_r[0] + part_r[1]
        acc = cc_r[...]
        for j in range(4):
            t = _dotd(co[:, 8 * j:8 * j + 8], ccr_r[8 * j:8 * j + 8, :])
            acc = acc + t * t
        h = acc
        for (v_r, c_r) in ((v1_r, c1_r), (v2_r, c2_r)):
            h = _dotd(h, v_r[...]) + c_r[...]
            h = _silu(_ln_rows(h))
        o = _dotd(h, v3_r[...]) + c3_r[...]
        if final:
            o = o * (sp_r[...] > -0.5).astype(jnp.float32)
        else:
            o = o + ns_r[...]
        o_r[...] = o

    grid = (npad // _BN,)
    fspec = lambda shape: pl.BlockSpec(shape, lambda i: tuple(0 for _ in shape))
    return pl.pallas_call(
        body,
        grid=grid,
        in_specs=[
            pl.BlockSpec((2, _BN, 32),
                         lambda i: (0, jnp.minimum(i, pblk_max), 0)),
            pl.BlockSpec((_BN, 64), lambda i: (i, 0)),
            fspec((32, 64)),
            fspec((64, 64)), fspec((1, 64)),
            fspec((64, 64)), fspec((1, 64)),
            fspec((64, kk)), fspec((1, kk)),
            pl.BlockSpec((_BN, 1), lambda i: (i, 0)),
            pl.BlockSpec((_BN, 8), lambda i: (i, 0)),
        ],
        out_specs=pl.BlockSpec((_BN, kk), lambda i: (i, 0)),
        out_shape=jax.ShapeDtypeStruct((npad, kk), jnp.float32),
    )(part, ccen, ccr, p0[0], p0[1], p1[0], p1[1], p2[0], p2[1], spf, nnsum)


def kernel(cart, atom_index, local_species, neigh_species, emb_neigh_params,
           emb_center_params, oc_params, out_params, contracted_coeff):
    n = local_species.shape[0]
    e = atom_index.shape[1]
    npad = ((n + _NW * _B - 1) // (_NW * _B)) * (_NW * _B)
    epad = ((e + _NW * _B - 1) // (_NW * _B)) * (_NW * _B)

    f32 = jnp.float32
    sp = local_species.astype(jnp.int32) + 1
    sp_pad = jnp.concatenate([sp, jnp.zeros((npad - n,), jnp.int32)])
    a0 = atom_index[0].astype(jnp.int32)
    a1 = atom_index[1].astype(jnp.int32)
    a0p = jnp.concatenate([a0, jnp.full((epad - e,), n, jnp.int32)])
    a1p = jnp.concatenate([a1, jnp.zeros((epad - e,), jnp.int32)])
    a0p2d = a0p.reshape(_NW, -1, _B)
    a1p2d = a1p.reshape(_NW, -1, _B)

    (w1n, b1n), (w2n, b2n) = emb_neigh_params
    (w1c, b1c), (w2c, b2c) = emb_center_params
    w1n_pad = jnp.concatenate(
        [w1n.astype(jnp.bfloat16).astype(f32),
         jnp.zeros((128 - w1n.shape[0], w1n.shape[1]), f32)])
    w1c_pad = jnp.concatenate(
        [w1c, jnp.zeros((128 - w1c.shape[0], w1c.shape[1]), f32)])

    cart4 = jnp.pad(cart, ((0, npad - n), (0, 1)))
    nf, ccen = _tc_node_feats(
        sp_pad.reshape(-1, 1), cart4, w1n_pad, w1c_pad,
        b1c.reshape(1, -1), w2c, b2c.reshape(1, -1))

    ga0, ga1 = _sc_gather([nf, nf], [a0p2d, a1p2d])

    dvt, feat = _tc_edge_dense(
        ga0.T, ga1.T, w2n.T, b1n.reshape(-1, 1), b2n.reshape(-1, 1))

    nacc = 10240
    zrows = jnp.zeros((nacc // _NS, 32), f32)
    ccr = contracted_coeff[:, jnp.asarray(_IDXP)]
    spf = sp_pad.astype(f32).reshape(-1, 1) - 1.0

    nn = jnp.zeros((npad, 8), f32)
    for r in range(OC_LOOP + 1):
        part = _sc_round(feat, nn, a0p2d, a1p2d, zrows, nacc)
        params = oc_params[r] if r < OC_LOOP else out_params
        nn = _tc_density_mlp(
            part, ccen, ccr[r].reshape(32, 64),
            (params[0][0], params[0][1].reshape(1, -1)),
            (params[1][0], params[1][1].reshape(1, -1)),
            (params[2][0], params[2][1].reshape(1, -1)),
            spf, nn, final=(r == OC_LOOP))

    dist_vec = dvt[:3, :e].T
    output = nn[:n]
    return (dist_vec, output)


# speedup vs baseline: 55.3483x; 1.0074x over previous
import functools

import jax
import jax.numpy as jnp
import numpy as np
from jax import lax
from jax.experimental import pallas as pl
from jax.experimental.pallas import tpu as pltpu
from jax.experimental.pallas import tpu_sc as plsc

NWAVE = 8
NORBIT = 64
OC_LOOP = 3
CUTOFF = 4.0
NUM_CLASSES = 118
_IDXP = np.array([0, 1, 1, 1])

_NC = 2
_NS = 16
_NW = _NC * _NS
_B = 128


def _mesh():
    return plsc.VectorSubcoreMesh(
        core_axis_name="c", subcore_axis_name="s", num_cores=_NC,
        num_subcores=_NS)


def _sc_gather(tables, idxs):
    ntab = len(tables)
    nblk = idxs[0].shape[1]
    npts = _NW * nblk * _B
    outs = tuple(
        jax.ShapeDtypeStruct((npts, int(t.shape[1])), jnp.float32)
        for t in tables)
    scratch = []
    for t in tables:
        scratch += [
            pltpu.VMEM((nblk, _B), jnp.int32),
            pltpu.VMEM((_B, int(t.shape[1])), jnp.float32),
            pltpu.SemaphoreType.DMA,
        ]

    nslot = 4
    for t in tables:
        for _ in range(nslot):
            scratch += [pltpu.VMEM((_B, int(t.shape[1])), jnp.float32),
                        pltpu.SemaphoreType.DMA,
                        pltpu.SemaphoreType.DMA]

    @functools.partial(pl.kernel, out_type=outs, mesh=_mesh(),
                       scratch_types=scratch,
                       compiler_params=pltpu.CompilerParams(
                           use_tc_tiling_on_sc=False))
    def k(*refs):
        tbl = refs[:ntab]
        idx = refs[ntab:2 * ntab]
        out = refs[2 * ntab:3 * ntab]
        sc = refs[3 * ntab:]
        idx_v = [sc[3 * t] for t in range(ntab)]
        ring = sc[3 * ntab:]

        def slot(t, l):
            o = 3 * (t * nslot + l)
            return ring[o], ring[o + 1], ring[o + 2]

        wid = lax.axis_index("s") * _NC + lax.axis_index("c")
        for t in range(ntab):
            pltpu.sync_copy(idx[t].at[wid], idx_v[t])

        def start_g(t, l, j):
            rb, gs, _ = slot(t, l)
            pltpu.async_copy(tbl[t].at[idx_v[t].at[j]], rb, gs)

        def wait_g(t, l, j):
            rb, gs, _ = slot(t, l)
            pltpu.make_async_copy(tbl[t].at[idx_v[t].at[j]], rb, gs).wait()

        def start_s(t, l, j):
            rb, _, ss = slot(t, l)
            base = wid * nblk * _B + j * _B
            pltpu.async_copy(rb, out[t].at[pl.ds(base, _B)], ss)

        def wait_s(t, l, j):
            rb, _, ss = slot(t, l)
            base = wid * nblk * _B + j * _B
            pltpu.make_async_copy(rb, out[t].at[pl.ds(base, _B)], ss).wait()

        for l in range(nslot):
            for t in range(ntab):
                start_g(t, l, l)

        def body(p, carry):
            for l in range(nslot):
                j = nslot * p + l
                for t in range(ntab):
                    wait_g(t, l, j)
                    start_s(t, l, j)
            for l in range(nslot):
                j = nslot * p + l
                jn = jnp.minimum(j + nslot, nblk - 1)
                for t in range(ntab):
                    wait_s(t, l, j)
                    start_g(t, l, jn)
            return carry

        lax.fori_loop(0, nblk // nslot, body, 0)
        for l in range(nslot):
            for t in range(ntab):
                wait_g(t, l, nblk - 1)

    res = k(*tables, *idxs)
    if not isinstance(res, (list, tuple)):
        res = [res]
    return list(res)


def _sc_round(feat, nnsum, idx0, idx1, zrows, nacc):
    nblk = idx0.shape[1]
    rpt = nacc // _NS
    epad = feat.shape[1]

    @functools.partial(
        pl.kernel,
        out_type=jax.ShapeDtypeStruct((_NC, nacc, 32), jnp.float32),
        mesh=_mesh(),
        scratch_types=[
            pltpu.VMEM((nblk, _B), jnp.int32),
            pltpu.VMEM((nblk, _B), jnp.int32),
            pltpu.VMEM((nacc, 8), jnp.float32),
            pltpu.VMEM((20, _B), jnp.float32),
            pltpu.VMEM((20, _B), jnp.float32),
            pltpu.VMEM((_B, 32), jnp.float32),
            pltpu.VMEM((_B, 32), jnp.float32),
            pltpu.SemaphoreType.DMA,
            pltpu.SemaphoreType.DMA,
            pltpu.SemaphoreType.DMA,
            pltpu.SemaphoreType.DMA,
            pltpu.VMEM_SHARED((nacc, 32), jnp.float32),
        ],
        compiler_params=pltpu.CompilerParams(
            use_tc_tiling_on_sc=False, needs_layout_passes=False))
    def k(f_h, nn_h, i0_h, i1_h, z_h, out_h,
          i0_v, i1_v, nn_v, fb0, fb1, pb0, pb1, sl0, sl1, ss0, ss1, acc):
        cid = lax.axis_index("c")
        sid = lax.axis_index("s")
        wid = sid * _NC + cid
        ebase = wid * nblk * _B

        def start_load(j, fb, sem):
            pltpu.async_copy(f_h.at[:, pl.ds(ebase + j * _B, _B)], fb, sem)

        def wait_load(j, fb, sem):
            pltpu.make_async_copy(
                f_h.at[:, pl.ds(ebase + j * _B, _B)], fb, sem).wait()

        def compute(j, fb, pb):
            for g in range(8):
                sl = pl.ds(g * 16, 16)
                e1 = i1_v[j, sl]
                erows = lax.iota(jnp.int32, 16) + (g * 16)
                for kw in range(8):
                    nk = plsc.load_gather(
                        nn_v, [e1, jnp.full((16,), kw, jnp.int32)])
                    rk = fb[kw, sl] * (fb[12 + kw, sl] + nk)
                    for ja in range(4):
                        plsc.store_scatter(
                            pb,
                            [erows, jnp.full((16,), ja * 8 + kw, jnp.int32)],
                            rk * fb[8 + ja, sl])

        def start_scatter(j, pb, sem):
            pltpu.async_copy(pb, acc.at[i0_v.at[j]], sem, add=True)

        def wait_scatter(j, pb, sem):
            pltpu.make_async_copy(pb, acc.at[i0_v.at[j]], sem).wait()

        pltpu.sync_copy(i0_h.at[wid], i0_v)
        pltpu.sync_copy(i1_h.at[wid], i1_v)
        start_load(0, fb0, sl0)
        pltpu.sync_copy(z_h, acc.at[pl.ds(sid * rpt, rpt)])
        pltpu.sync_copy(nn_h.at[pl.ds(0, nacc)], nn_v)
        plsc.subcore_barrier()

        def body(p, carry):
            j0 = 2 * p
            j1 = 2 * p + 1
            start_load(j1, fb1, sl1)
            wait_load(j0, fb0, sl0)

            @pl.when(p > 0)
            def _():
                wait_scatter(j0 - 2, pb0, ss0)
            compute(j0, fb0, pb0)
            start_scatter(j0, pb0, ss0)
            start_load(jnp.minimum(j0 + 2, nblk - 1), fb0, sl0)
            wait_load(j1, fb1, sl1)

            @pl.when(p > 0)
            def _():
                wait_scatter(j1 - 2, pb1, ss1)
            compute(j1, fb1, pb1)
            start_scatter(j1, pb1, ss1)
            return carry

        lax.fori_loop(0, nblk // 2, body, 0)
        wait_load(nblk - 1, fb0, sl0)
        wait_scatter(nblk - 2, pb0, ss0)
        wait_scatter(nblk - 1, pb1, ss1)
        plsc.subcore_barrier()
        pltpu.sync_copy(acc.at[pl.ds(sid * rpt, rpt)],
                        out_h.at[cid, pl.ds(sid * rpt, rpt)])

    return k(feat, nnsum, idx0, idx1, zrows)


def _ln_cols(x):
    m = jnp.mean(x, axis=0, keepdims=True)
    v = jnp.mean(jnp.square(x - m), axis=0, keepdims=True)
    return (x - m) / jnp.sqrt(v + 1e-5)


def _ln_rows(x):
    m = jnp.mean(x, axis=1, keepdims=True)
    v = jnp.mean(jnp.square(x - m), axis=1, keepdims=True)
    return (x - m) / jnp.sqrt(v + 1e-5)


def _silu(x):
    return x * jax.nn.sigmoid(x)


def _rb(x):
    return x.astype(jnp.bfloat16).astype(jnp.float32)


def _dotd(a, b):
    return jnp.dot(_rb(a), _rb(b), preferred_element_type=jnp.float32,
                   precision=lax.Precision.HIGHEST)


_BN = 1024


def _tc_node_feats(spi, cart4, w1nr, w1c, b1c, w2c, b2c):
    npad = spi.shape[0]

    def body(sp_r, c_r, w1n_r, w1c_r, b1c_r, w2c_r, b2c_r, nf_o, cc_o):
        ids = lax.broadcasted_iota(jnp.int32, (_BN, 128), 1)
        oh = (ids == sp_r[...]).astype(jnp.float32)
        emb8 = jnp.dot(oh, w1n_r[...], preferred_element_type=jnp.float32,
                       precision=lax.Precision.HIGHEST)
        htbl = _silu(_ln_rows(_rb(w1c_r[...]) + b1c_r[...]))
        ctbl = _dotd(htbl, w2c_r[...]) + b2c_r[...]
        cc_o[...] = jnp.dot(oh, ctbl, preferred_element_type=jnp.float32,
                            precision=lax.Precision.HIGHEST)
        nf_o[...] = jnp.concatenate(
            [c_r[...], jnp.zeros((_BN, 4), jnp.float32), emb8], axis=1)

    grid = (npad // _BN,)
    fspec = lambda shape: pl.BlockSpec(shape, lambda i: tuple(0 for _ in shape))
    return pl.pallas_call(
        body,
        grid=grid,
        in_specs=[
            pl.BlockSpec((_BN, 1), lambda i: (i, 0)),
            pl.BlockSpec((_BN, 4), lambda i: (i, 0)),
            fspec((128, 8)), fspec((128, 8)), fspec((1, 8)),
            fspec((8, NORBIT)), fspec((1, NORBIT)),
        ],
        out_specs=[pl.BlockSpec((_BN, 16), lambda i: (i, 0)),
                   pl.BlockSpec((_BN, NORBIT), lambda i: (i, 0))],
        out_shape=[jax.ShapeDtypeStruct((npad, 16), jnp.float32),
                   jax.ShapeDtypeStruct((npad, NORBIT), jnp.float32)],
    )(spi, cart4, w1nr, w1c, b1c, w2c, b2c)


_BE = 2048


def _tc_edge_dense(ga0t, ga1t, w2t, b1, b2t):
    epad = ga0t.shape[1]

    def body(a_r, b_r, w2_r, b1_r, b2_r, dv_o, f_o):
        a = a_r[...]
        b = b_r[...]
        dv = a[0:3, :] - b[0:3, :]
        d2 = jnp.sum(dv * dv, axis=0, keepdims=True)
        dist = jnp.sqrt(d2)
        e = a[8:16, :] + b[8:16, :] + b1_r[...]
        h = _silu(_ln_cols(e))
        ne = _dotd(w2_r[...], h) + b2_r[...]
        w = ne[0:8, :]
        rad = jnp.exp(-jnp.square(ne[8:16, :] * (dist - ne[16:24, :])))
        cut = jnp.square(0.5 * jnp.cos(dist * (np.pi / CUTOFF)) + 0.5)
        ang = jnp.concatenate([cut, cut * dv], axis=0)
        dv_o[...] = jnp.concatenate([dv, dist], axis=0)
        f_o[...] = jnp.concatenate([rad, ang, w], axis=0)

    grid = (epad // _BE,)
    espec = lambda r: pl.BlockSpec((r, _BE), lambda i: (0, i))
    fspec = lambda shape: pl.BlockSpec(shape, lambda i: (0, 0))
    return pl.pallas_call(
        body,
        grid=grid,
        in_specs=[espec(16), espec(16), fspec((24, 8)), fspec((8, 1)),
                  fspec((24, 1))],
        out_specs=[espec(4), espec(20)],
        out_shape=[
            jax.ShapeDtypeStruct((4, epad), jnp.float32),
            jax.ShapeDtypeStruct((20, epad), jnp.float32),
        ],
    )(ga0t, ga1t, w2t, b1, b2t)


def _tc_density_mlp(part, ccen, ccr, p0, p1, p2, spf, nnsum, final):
    npad = ccen.shape[0]
    kk = p2[0].shape[1]
    pblk_max = part.shape[1] // _BN - 1

    def body(part_r, cc_r, ccr_r, v1_r, c1_r, v2_r, c2_r, v3_r, c3_r, sp_r,
             ns_r, o_r):
        co = part_r[0] + part_r[1]
        acc = cc_r[...]
        for j in range(4):
            t = _dotd(co[:, 8 * j:8 * j + 8], ccr_r[8 * j:8 * j + 8, :])
            acc = acc + t * t
        h = acc
        for (v_r, c_r) in ((v1_r, c1_r), (v2_r, c2_r)):
            h = _dotd(h, v_r[...]) + c_r[...]
            h = _silu(_ln_rows(h))
        o = _dotd(h, v3_r[...]) + c3_r[...]
        if final:
            o = o * (sp_r[...] > -0.5).astype(jnp.float32)
        else:
            o = o + ns_r[...]
        o_r[...] = o

    grid = (npad // _BN,)
    fspec = lambda shape: pl.BlockSpec(shape, lambda i: tuple(0 for _ in shape))
    return pl.pallas_call(
        body,
        grid=grid,
        in_specs=[
            pl.BlockSpec((2, _BN, 32),
                         lambda i: (0, jnp.minimum(i, pblk_max), 0)),
            pl.BlockSpec((_BN, 64), lambda i: (i, 0)),
            fspec((32, 64)),
            fspec((64, 64)), fspec((1, 64)),
            fspec((64, 64)), fspec((1, 64)),
            fspec((64, kk)), fspec((1, kk)),
            pl.BlockSpec((_BN, 1), lambda i: (i, 0)),
            pl.BlockSpec((_BN, 8), lambda i: (i, 0)),
        ],
        out_specs=pl.BlockSpec((_BN, kk), lambda i: (i, 0)),
        out_shape=jax.ShapeDtypeStruct((npad, kk), jnp.float32),
    )(part, ccen, ccr, p0[0], p0[1], p1[0], p1[1], p2[0], p2[1], spf, nnsum)


def kernel(cart, atom_index, local_species, neigh_species, emb_neigh_params,
           emb_center_params, oc_params, out_params, contracted_coeff):
    n = local_species.shape[0]
    e = atom_index.shape[1]
    npad = ((n + _NW * _B - 1) // (_NW * _B)) * (_NW * _B)
    epad = ((e + _NW * _B - 1) // (_NW * _B)) * (_NW * _B)

    f32 = jnp.float32
    sp = local_species.astype(jnp.int32) + 1
    sp_pad = jnp.concatenate([sp, jnp.zeros((npad - n,), jnp.int32)])
    a0 = atom_index[0].astype(jnp.int32)
    a1 = atom_index[1].astype(jnp.int32)
    a0p = jnp.concatenate([a0, jnp.full((epad - e,), n, jnp.int32)])
    a1p = jnp.concatenate([a1, jnp.zeros((epad - e,), jnp.int32)])
    a0p2d = a0p.reshape(_NW, -1, _B)
    a1p2d = a1p.reshape(_NW, -1, _B)

    (w1n, b1n), (w2n, b2n) = emb_neigh_params
    (w1c, b1c), (w2c, b2c) = emb_center_params
    w1n_pad = jnp.concatenate(
        [w1n.astype(jnp.bfloat16).astype(f32),
         jnp.zeros((128 - w1n.shape[0], w1n.shape[1]), f32)])
    w1c_pad = jnp.concatenate(
        [w1c, jnp.zeros((128 - w1c.shape[0], w1c.shape[1]), f32)])

    cart4 = jnp.pad(cart, ((0, npad - n), (0, 1)))
    nf, ccen = _tc_node_feats(
        sp_pad.reshape(-1, 1), cart4, w1n_pad, w1c_pad,
        b1c.reshape(1, -1), w2c, b2c.reshape(1, -1))

    ga0, ga1 = _sc_gather([nf, nf], [a0p2d, a1p2d])

    dvt, feat = _tc_edge_dense(
        ga0.T, ga1.T, w2n.T, b1n.reshape(-1, 1), b2n.reshape(-1, 1))

    nacc = 10240
    zrows = jnp.zeros((nacc // _NS, 32), f32)
    ccr = contracted_coeff[:, jnp.asarray(_IDXP)]
    spf = sp_pad.astype(f32).reshape(-1, 1) - 1.0

    nn = jnp.zeros((npad, 8), f32)
    for r in range(OC_LOOP + 1):
        part = _sc_round(feat, nn, a0p2d, a1p2d, zrows, nacc)
        params = oc_params[r] if r < OC_LOOP else out_params
        nn = _tc_density_mlp(
            part, ccen, ccr[r].reshape(32, 64),
            (params[0][0], params[0][1].reshape(1, -1)),
            (params[1][0], params[1][1].reshape(1, -1)),
            (params[2][0], params[2][1].reshape(1, -1)),
            spf, nn, final=(r == OC_LOOP))

    dist_vec = dvt[:3, :e].T
    output = nn[:n]
    return (dist_vec, output)
